# baseline (speedup 1.0000x reference)
import jax
import jax.numpy as jnp
from jax import lax
from jax.experimental import pallas as pl
from jax.experimental.pallas import tpu as pltpu
from jax.experimental.pallas import tpu_sc as plsc

B, F, IMG = 4, 384, 128 * 128
L, K = 64 * 64, 9
BF = B * F
NC, NS, LANES = 2, 16, 16
NW = NC * NS
ROWS = BF // NW
NPAIRS = ROWS // 2
NLB = L // LANES
SENT = IMG
RBUF = IMG + 128
MCH = (K * L) // 3
PACK = plsc.PackFormat.INTERLEAVED


def _pool_body(img, idxr, maskr, out,
               enc_v, r00, r01, r10, r11, o00, o01, o10, o11,
               isem0, isem1, osem0, osem1):
    wid = lax.axis_index("s") * NC + lax.axis_index("c")
    base = wid * ROWS

    rbufs = (r00, r01, r10, r11)
    sent = jnp.full((LANES,), SENT, jnp.int32)

    def _build(xbuf, mbuf, fbase):
        @plsc.parallel_loop(0, MCH // 32, unroll=2)
        def _b(i, fbase=fbase, xbuf=xbuf, mbuf=mbuf):
            off = i * 32
            ia = plsc.bitcast(xbuf[0, pl.ds(off, LANES)], jnp.int32)
            ib = plsc.bitcast(xbuf[0, pl.ds(off + LANES, LANES)], jnp.int32)
            ma = mbuf[0, pl.ds(off, LANES)]
            mb = mbuf[0, pl.ds(off + LANES, LANES)]
            ea = jnp.where(ma > 0.0, ia, sent)
            eb = jnp.where(mb > 0.0, ib, sent)
            packed = plsc.pack(ea, eb, format=PACK)
            enc_v[pl.ds((fbase + off) // 2, LANES)] = plsc.bitcast(
                packed, jnp.int32)

    for c in range(2):
        pltpu.async_copy(idxr.at[pl.ds(c * MCH, MCH)],
                         rbufs[c].at[0, pl.ds(0, MCH)], isem0)
        pltpu.async_copy(maskr.at[pl.ds(c * MCH, MCH)],
                         rbufs[2 + c].at[0, pl.ds(0, MCH)], isem0)
    for c in range(2):
        pltpu.make_async_copy(idxr.at[pl.ds(c * MCH, MCH)],
                              rbufs[c].at[0, pl.ds(0, MCH)], isem0).wait()
        pltpu.make_async_copy(maskr.at[pl.ds(c * MCH, MCH)],
                              rbufs[2 + c].at[0, pl.ds(0, MCH)],
                              isem0).wait()
    _build(r00, r10, 0)
    _build(r01, r11, MCH)
    pltpu.async_copy(idxr.at[pl.ds(2 * MCH, MCH)],
                     r00.at[0, pl.ds(0, MCH)], isem0)
    pltpu.async_copy(maskr.at[pl.ds(2 * MCH, MCH)],
                     r01.at[0, pl.ds(0, MCH)], isem0)
    pltpu.make_async_copy(idxr.at[pl.ds(2 * MCH, MCH)],
                          r00.at[0, pl.ds(0, MCH)], isem0).wait()
    pltpu.make_async_copy(maskr.at[pl.ds(2 * MCH, MCH)],
                          r01.at[0, pl.ds(0, MCH)], isem0).wait()
    _build(r00, r01, 2 * MCH)

    zeros = jnp.zeros((LANES,), jnp.float32)
    for rb in rbufs:
        rb[0, pl.ds(IMG, LANES)] = zeros

    pltpu.async_copy(img.at[pl.ds(base + 0, 1), :],
                     r00.at[:, pl.ds(0, IMG)], isem0)
    pltpu.async_copy(img.at[pl.ds(base + 1, 1), :],
                     r01.at[:, pl.ds(0, IMG)], isem0)
    pltpu.async_copy(img.at[pl.ds(base + 2, 1), :],
                     r10.at[:, pl.ds(0, IMG)], isem1)
    pltpu.async_copy(img.at[pl.ds(base + 3, 1), :],
                     r11.at[:, pl.ds(0, IMG)], isem1)

    sets = ((r00, r01, o00, o01, isem0, osem0),
            (r10, r11, o10, o11, isem1, osem1))

    @pl.loop(0, NPAIRS, step=2)
    def _pair2(p0):
        for b, (ra, rb, oa, ob, isem, osem) in enumerate(sets):
            p = p0 + b
            rowa = base + 2 * p
            pltpu.make_async_copy(img.at[pl.ds(rowa, 1), :],
                                  ra.at[:, pl.ds(0, IMG)], isem).wait()
            pltpu.make_async_copy(img.at[pl.ds(rowa + 1, 1), :],
                                  rb.at[:, pl.ds(0, IMG)], isem).wait()

            @pl.when(p0 >= 2)
            def _drain():
                pltpu.make_async_copy(oa, out.at[pl.ds(rowa, 1), :],
                                      osem).wait()
                pltpu.make_async_copy(ob, out.at[pl.ds(rowa, 1), :],
                                      osem).wait()

            ra1, rb1 = ra.at[0], rb.at[0]

            @plsc.parallel_loop(0, NLB, step=4, unroll=1)
            def _lb(lb):
                off = lb * LANES
                acc = [None] * 8
                for k in range(K):
                    kb = k * (L // 2) + off // 2
                    pA = enc_v[pl.ds(kb, LANES)]
                    pB = enc_v[pl.ds(kb + LANES, LANES)]
                    e0, e1 = plsc.unpack(plsc.bitcast(pA, jnp.int16),
                                         format=PACK,
                                         preferred_element_type=jnp.int32)
                    e2, e3 = plsc.unpack(plsc.bitcast(pB, jnp.int16),
                                         format=PACK,
                                         preferred_element_type=jnp.int32)
                    for j, e in enumerate((e0, e1, e2, e3)):
                        g = plsc.load_gather(ra1, [e])
                        h = plsc.load_gather(rb1, [e])
                        acc[j] = g if acc[j] is None else acc[j] + g
                        acc[4 + j] = h if acc[4 + j] is None else acc[4 + j] + h
                for j in range(4):
                    slj = pl.ds(off + j * LANES, LANES)
                    oa[0, slj] = acc[j] * (1.0 / K)
                    ob[0, slj] = acc[4 + j] * (1.0 / K)

            pltpu.async_copy(oa, out.at[pl.ds(rowa, 1), :], osem)
            pltpu.async_copy(ob, out.at[pl.ds(rowa + 1, 1), :], osem)

            @pl.when(p0 < NPAIRS - 2)
            def _prefetch():
                na = rowa + 4
                pltpu.async_copy(img.at[pl.ds(na, 1), :],
                                 ra.at[:, pl.ds(0, IMG)], isem)
                pltpu.async_copy(img.at[pl.ds(na + 1, 1), :],
                                 rb.at[:, pl.ds(0, IMG)], isem)

    for (_, _, oa, ob, _, osem) in sets:
        pltpu.make_async_copy(oa, out.at[pl.ds(base, 1), :], osem).wait()
        pltpu.make_async_copy(ob, out.at[pl.ds(base, 1), :], osem).wait()


@jax.jit
def _pool(img2d, idx1d, mask1d):
    fn = pl.kernel(
        _pool_body,
        out_type=jax.ShapeDtypeStruct((BF, L), jnp.float32),
        mesh=plsc.VectorSubcoreMesh(core_axis_name="c", subcore_axis_name="s"),
        compiler_params=pltpu.CompilerParams(needs_layout_passes=False),
        scratch_types=[
            pltpu.VMEM((K * L // 2,), jnp.int32),
            pltpu.VMEM((1, RBUF), jnp.float32),
            pltpu.VMEM((1, RBUF), jnp.float32),
            pltpu.VMEM((1, RBUF), jnp.float32),
            pltpu.VMEM((1, RBUF), jnp.float32),
            pltpu.VMEM((1, L), jnp.float32),
            pltpu.VMEM((1, L), jnp.float32),
            pltpu.VMEM((1, L), jnp.float32),
            pltpu.VMEM((1, L), jnp.float32),
            pltpu.SemaphoreType.DMA,
            pltpu.SemaphoreType.DMA,
            pltpu.SemaphoreType.DMA,
            pltpu.SemaphoreType.DMA,
        ],
    )
    return fn(img2d, idx1d, mask1d)


def kernel(input_images, indices, mask):
    out2d = _pool(input_images.reshape(BF, IMG),
                  lax.bitcast_convert_type(indices, jnp.float32)
                     .reshape(K * L),
                  mask.reshape(K * L))
    return out2d.reshape(B, F, L)


# speedup vs baseline: 1.0558x; 1.0558x over previous
import jax
import jax.numpy as jnp
from jax import lax
from jax.experimental import pallas as pl
from jax.experimental.pallas import tpu as pltpu
from jax.experimental.pallas import tpu_sc as plsc

B, F, IMG = 4, 384, 128 * 128
L, K = 64 * 64, 9
BF = B * F
NC, NS, LANES = 2, 16, 16
NW = NC * NS
ROWS = BF // NW
NPAIRS = ROWS // 2
NLB = L // LANES
SENT = IMG
RBUF = IMG + 128
MCH = (K * L) // 3
PACK = plsc.PackFormat.INTERLEAVED


def _pool_body(img, idxr, maskr, out,
               enc_v, r00, r01, r10, r11, o00, o01, o10, o11,
               isem0, isem1, osem0, osem1):
    wid = lax.axis_index("s") * NC + lax.axis_index("c")
    base = wid * ROWS

    rbufs = (r00, r01, r10, r11)
    sent = jnp.full((LANES,), SENT, jnp.int32)

    def _build(xbuf, mbuf, fbase):
        @plsc.parallel_loop(0, MCH // 32, unroll=2)
        def _b(i, fbase=fbase, xbuf=xbuf, mbuf=mbuf):
            off = i * 32
            ia = plsc.bitcast(xbuf[0, pl.ds(off, LANES)], jnp.int32)
            ib = plsc.bitcast(xbuf[0, pl.ds(off + LANES, LANES)], jnp.int32)
            ma = mbuf[0, pl.ds(off, LANES)]
            mb = mbuf[0, pl.ds(off + LANES, LANES)]
            ea = jnp.where(ma > 0.0, ia, sent)
            eb = jnp.where(mb > 0.0, ib, sent)
            packed = plsc.pack(ea, eb, format=PACK)
            enc_v[pl.ds((fbase + off) // 2, LANES)] = plsc.bitcast(
                packed, jnp.int32)

    pltpu.async_copy(img.at[pl.ds(base + 0, 1), :],
                     r10.at[:, pl.ds(0, IMG)], isem1)
    pltpu.async_copy(img.at[pl.ds(base + 1, 1), :],
                     r11.at[:, pl.ds(0, IMG)], isem1)
    for c in range(3):
        pltpu.async_copy(idxr.at[pl.ds(c * MCH, MCH)],
                         r00.at[0, pl.ds(0, MCH)], isem0)
        pltpu.async_copy(maskr.at[pl.ds(c * MCH, MCH)],
                         r01.at[0, pl.ds(0, MCH)], isem0)
        pltpu.make_async_copy(idxr.at[pl.ds(c * MCH, MCH)],
                              r00.at[0, pl.ds(0, MCH)], isem0).wait()
        pltpu.make_async_copy(maskr.at[pl.ds(c * MCH, MCH)],
                              r01.at[0, pl.ds(0, MCH)], isem0).wait()
        _build(r00, r01, c * MCH)

    zeros = jnp.zeros((LANES,), jnp.float32)
    for rb in rbufs:
        rb[0, pl.ds(IMG, LANES)] = zeros

    pltpu.async_copy(img.at[pl.ds(base + 2, 1), :],
                     r00.at[:, pl.ds(0, IMG)], isem0)
    pltpu.async_copy(img.at[pl.ds(base + 3, 1), :],
                     r01.at[:, pl.ds(0, IMG)], isem0)

    sets = ((r10, r11, o10, o11, isem1, osem1),
            (r00, r01, o00, o01, isem0, osem0))

    @pl.loop(0, NPAIRS, step=2)
    def _pair2(p0):
        for b, (ra, rb, oa, ob, isem, osem) in enumerate(sets):
            p = p0 + b
            rowa = base + 2 * p
            pltpu.make_async_copy(img.at[pl.ds(rowa, 1), :],
                                  ra.at[:, pl.ds(0, IMG)], isem).wait()
            pltpu.make_async_copy(img.at[pl.ds(rowa + 1, 1), :],
                                  rb.at[:, pl.ds(0, IMG)], isem).wait()

            @pl.when(p0 >= 2)
            def _drain():
                pltpu.make_async_copy(oa, out.at[pl.ds(rowa, 1), :],
                                      osem).wait()
                pltpu.make_async_copy(ob, out.at[pl.ds(rowa, 1), :],
                                      osem).wait()

            ra1, rb1 = ra.at[0], rb.at[0]

            @plsc.parallel_loop(0, NLB, step=2, unroll=1)
            def _lb(lb):
                off = lb * LANES
                sl0 = pl.ds(off, LANES)
                sl1 = pl.ds(off + LANES, LANES)
                a0 = a1 = b0 = b1 = None
                for k in range(K):
                    pk32 = enc_v[pl.ds(k * (L // 2) + off // 2, LANES)]
                    e0, e1 = plsc.unpack(plsc.bitcast(pk32, jnp.int16),
                                         format=PACK,
                                         preferred_element_type=jnp.int32)
                    ga0 = plsc.load_gather(ra1, [e0])
                    ga1 = plsc.load_gather(ra1, [e1])
                    gb0 = plsc.load_gather(rb1, [e0])
                    gb1 = plsc.load_gather(rb1, [e1])
                    a0 = ga0 if a0 is None else a0 + ga0
                    a1 = ga1 if a1 is None else a1 + ga1
                    b0 = gb0 if b0 is None else b0 + gb0
                    b1 = gb1 if b1 is None else b1 + gb1
                oa[0, sl0] = a0 * (1.0 / K)
                oa[0, sl1] = a1 * (1.0 / K)
                ob[0, sl0] = b0 * (1.0 / K)
                ob[0, sl1] = b1 * (1.0 / K)

            pltpu.async_copy(oa, out.at[pl.ds(rowa, 1), :], osem)
            pltpu.async_copy(ob, out.at[pl.ds(rowa + 1, 1), :], osem)

            @pl.when(p0 < NPAIRS - 2)
            def _prefetch():
                na = rowa + 4
                pltpu.async_copy(img.at[pl.ds(na, 1), :],
                                 ra.at[:, pl.ds(0, IMG)], isem)
                pltpu.async_copy(img.at[pl.ds(na + 1, 1), :],
                                 rb.at[:, pl.ds(0, IMG)], isem)

    for (_, _, oa, ob, _, osem) in sets:
        pltpu.make_async_copy(oa, out.at[pl.ds(base, 1), :], osem).wait()
        pltpu.make_async_copy(ob, out.at[pl.ds(base, 1), :], osem).wait()


@jax.jit
def _pool(img2d, idx1d, mask1d):
    fn = pl.kernel(
        _pool_body,
        out_type=jax.ShapeDtypeStruct((BF, L), jnp.float32),
        mesh=plsc.VectorSubcoreMesh(core_axis_name="c", subcore_axis_name="s"),
        compiler_params=pltpu.CompilerParams(needs_layout_passes=False),
        scratch_types=[
            pltpu.VMEM((K * L // 2,), jnp.int32),
            pltpu.VMEM((1, RBUF), jnp.float32),
            pltpu.VMEM((1, RBUF), jnp.float32),
            pltpu.VMEM((1, RBUF), jnp.float32),
            pltpu.VMEM((1, RBUF), jnp.float32),
            pltpu.VMEM((1, L), jnp.float32),
            pltpu.VMEM((1, L), jnp.float32),
            pltpu.VMEM((1, L), jnp.float32),
            pltpu.VMEM((1, L), jnp.float32),
            pltpu.SemaphoreType.DMA,
            pltpu.SemaphoreType.DMA,
            pltpu.SemaphoreType.DMA,
            pltpu.SemaphoreType.DMA,
        ],
    )
    return fn(img2d, idx1d, mask1d)


def kernel(input_images, indices, mask):
    out2d = _pool(input_images.reshape(BF, IMG),
                  lax.bitcast_convert_type(indices, jnp.float32)
                     .reshape(K * L),
                  mask.reshape(K * L))
    return out2d.reshape(B, F, L)
